# baseline (speedup 1.0000x reference)
import functools
import math

import jax
import jax.numpy as jnp
from jax import lax
from jax.experimental import pallas as pl
from jax.experimental.pallas import tpu as pltpu
from jax.experimental.pallas import tpu_sc as plsc

_VOCAB = 22
_D_OUT = 56
_NC = 2
_NS = 16
_NW = _NC * _NS
_LANE = 128
_RGRPS = 8
_CGRPS = _NW // _RGRPS
_DT = _D_OUT // 8


def _table_body(aa_ref, gt_ref, g_ids_ref, props_ref, w1t_ref, b1_ref,
                ln1g_ref, ln1b_ref, w2t_ref, b2_ref, ng_ref, nb_ref, out_ref):
    f32 = jnp.float32
    aa = aa_ref[...]
    gids = g_ids_ref[...]
    onehot = (gids == lax.broadcasted_iota(jnp.int32, (_VOCAB, 5), 1)).astype(f32)
    group_emb = jnp.dot(onehot, gt_ref[...], preferred_element_type=f32)

    h = jnp.dot(props_ref[...], w1t_ref[...], preferred_element_type=f32)
    h = h + b1_ref[...]
    mean = jnp.mean(h, axis=1, keepdims=True)
    var = jnp.mean((h - mean) ** 2, axis=1, keepdims=True)
    h = (h - mean) * lax.rsqrt(var + 1e-5) * ln1g_ref[...] + ln1b_ref[...]
    h = 0.5 * h * (1.0 + lax.erf(h / math.sqrt(2.0)))
    prop_emb = jnp.dot(h, w2t_ref[...], preferred_element_type=f32) + b2_ref[...]

    comb = jnp.concatenate([aa, group_emb, prop_emb], axis=1)
    mean2 = jnp.mean(comb, axis=1, keepdims=True)
    var2 = jnp.mean((comb - mean2) ** 2, axis=1, keepdims=True)
    fused = ((comb - mean2) * lax.rsqrt(var2 + 1e-5) * ng_ref[...]
             + nb_ref[...])

    eye = (lax.broadcasted_iota(jnp.int32, (_D_OUT, _D_OUT), 0)
           == lax.broadcasted_iota(jnp.int32, (_D_OUT, _D_OUT), 1)).astype(f32)
    fused_t = lax.dot_general(eye, fused, (((1,), (1,)), ((), ())),
                              precision=lax.Precision.HIGHEST,
                              preferred_element_type=f32)
    place = (lax.broadcasted_iota(jnp.int32, (_VOCAB, _LANE), 0)
             == lax.broadcasted_iota(jnp.int32, (_VOCAB, _LANE), 1)).astype(f32)
    out_ref[...] = jnp.dot(fused_t, place, precision=lax.Precision.HIGHEST,
                           preferred_element_type=f32)


def _build_table(aa_table, group_table, aa_to_group, aa_properties,
                 W1, b1, ln1_g, ln1_b, W2, b2, norm_g, norm_b):
    return pl.pallas_call(
        _table_body,
        out_shape=jax.ShapeDtypeStruct((_D_OUT, _LANE), jnp.float32),
    )(aa_table, group_table, aa_to_group.reshape(_VOCAB, 1).astype(jnp.int32),
      aa_properties, W1.T, b1.reshape(1, -1), ln1_g.reshape(1, -1),
      ln1_b.reshape(1, -1), W2.T, b2.reshape(1, -1), norm_g.reshape(1, -1),
      norm_b.reshape(1, -1))


def _gather_body(table_hbm, idx_hbm, out_hbm, table_v, idx_v, st0, st1,
                 wsem0, wsem1):
    n_cols = idx_hbm.shape[0]
    n_rt = idx_hbm.shape[1]
    cols_w = n_cols // _CGRPS
    rt_w = n_rt // _RGRPS
    wid = lax.axis_index("s") * _NC + lax.axis_index("c")
    rgrp = wid // _CGRPS
    cgrp = wid % _CGRPS
    c0 = cgrp * cols_w
    rt0 = rgrp * rt_w

    pltpu.sync_copy(table_hbm, table_v)
    pltpu.sync_copy(idx_hbm.at[pl.ds(c0, cols_w), pl.ds(rt0, rt_w)], idx_v)

    def compute(cl, st):
        for rtl in range(rt_w):
            def grp_loop(g, carry2, _rtl=rtl):
                ids = idx_v[cl, _rtl, pl.ds(g * 16, 16)]
                vec = ids
                st[0, _rtl * 8, pl.ds(g * 16, 16)] = (
                    plsc.load_gather(table_v, [vec]))
                for d in range(1, _D_OUT):
                    vec = vec + 128
                    st[d // 8, _rtl * 8 + (d % 8), pl.ds(g * 16, 16)] = (
                        plsc.load_gather(table_v, [vec]))
                return carry2
            lax.fori_loop(0, 8, grp_loop, 0)

    def start_writes(cl, st, wsem):
        c = c0 + cl
        for dt in range(_DT):
            row = (c * _DT + dt) * (n_rt * 8) + rt0 * 8
            pltpu.async_copy(st.at[dt], out_hbm.at[pl.ds(row, rt_w * 8)], wsem)

    def drain_writes(st, wsem):
        for dt in range(_DT):
            pltpu.make_async_copy(out_hbm.at[pl.ds(0, rt_w * 8)],
                                  st.at[dt], wsem).wait()

    def outer(cp, carry):
        cl0 = cp * 2

        @pl.when(cp >= 1)
        def _():
            drain_writes(st0, wsem0)
        compute(cl0, st0)
        start_writes(cl0, st0, wsem0)

        @pl.when(cp >= 1)
        def _():
            drain_writes(st1, wsem1)
        compute(cl0 + 1, st1)
        start_writes(cl0 + 1, st1, wsem1)
        return carry

    lax.fori_loop(0, cols_w // 2, outer, 0)
    drain_writes(st0, wsem0)
    drain_writes(st1, wsem1)


def _gather(table, idx3, n_out_rows):
    n_cols, n_rt, _ = idx3.shape
    cols_w = n_cols // _CGRPS
    rt_w = n_rt // _RGRPS
    mesh = plsc.VectorSubcoreMesh(core_axis_name="c", subcore_axis_name="s")
    return pl.kernel(
        _gather_body,
        out_type=jax.ShapeDtypeStruct((n_out_rows, _LANE), jnp.float32),
        mesh=mesh,
        scratch_types=[
            pltpu.VMEM((_D_OUT * _LANE,), jnp.float32),
            pltpu.VMEM((cols_w, rt_w, _LANE), jnp.int32),
            pltpu.VMEM((_DT, rt_w * 8, _LANE), jnp.float32),
            pltpu.VMEM((_DT, rt_w * 8, _LANE), jnp.float32),
            pltpu.SemaphoreType.DMA,
            pltpu.SemaphoreType.DMA,
        ],
        compiler_params=pltpu.CompilerParams(use_tc_tiling_on_sc=False,
                                             needs_layout_passes=False),
    )(table, idx3)


def kernel(token_indices, aa_table, group_table, aa_to_group, aa_properties,
           W1, b1, ln1_g, ln1_b, W2, b2, norm_g, norm_b):
    n_rows, n_cols = token_indices.shape
    n_rt = n_rows // _LANE
    table = _build_table(aa_table, group_table, aa_to_group, aa_properties,
                         W1, b1, ln1_g, ln1_b, W2, b2, norm_g, norm_b)
    idx3 = token_indices.T.reshape(n_cols, n_rt, _LANE).astype(jnp.int32)
    n_out_rows = n_cols * _DT * n_rt * 8
    out2 = _gather(table.reshape(_D_OUT * _LANE), idx3, n_out_rows)
    out5 = out2.reshape(n_cols, _DT, n_rt, 8, _LANE)
    return out5.transpose(2, 4, 0, 1, 3).reshape(n_rows, n_cols, _D_OUT)
